# baseline (speedup 1.0000x reference)
import functools

import jax
import jax.numpy as jnp
from jax import lax
from jax.experimental import pallas as pl
from jax.experimental.pallas import tpu as pltpu
from jax.experimental.pallas import tpu_sc as plsc

L = 16


def _make_sc_call(N, C):
    assert C % 8 == 0 and N % 128 == 0 and N % L == 0
    rows_per_worker = N // L

    mesh = plsc.VectorSubcoreMesh(
        core_axis_name="c", subcore_axis_name="s", num_cores=1, num_subcores=L
    )

    @functools.partial(
        pl.kernel,
        out_type=jax.ShapeDtypeStruct((L * L,), jnp.float32),
        mesh=mesh,
        scratch_types=[
            pltpu.VMEM((rows_per_worker,), jnp.int32),
            pltpu.VMEM((rows_per_worker,), jnp.float32),
            pltpu.VMEM((rows_per_worker,), jnp.int32),
            pltpu.VMEM((rows_per_worker,), jnp.float32),
            pltpu.VMEM((L,), jnp.float32),
            pltpu.SemaphoreType.DMA,
            pltpu.SemaphoreType.DMA,
            pltpu.SemaphoreType.DMA,
        ],
        compiler_params=pltpu.CompilerParams(skip_device_barrier=True),
    )
    def sc_fn(prob_hbm, tgt_hbm, rew_hbm, out_hbm,
              tgt_v, rew_v, idx_v, vals_v, part_v, sem, sem_t, sem_r):
        sid = lax.axis_index("s")
        base = sid * rows_per_worker

        tgt_cp = pltpu.async_copy(
            tgt_hbm.at[pl.ds(base, rows_per_worker)], tgt_v, sem_t)
        rew_cp = pltpu.async_copy(
            rew_hbm.at[pl.ds(base, rows_per_worker)], rew_v, sem_r)
        tgt_cp.wait()

        iota = lax.iota(jnp.int32, L)
        for g in range(rows_per_worker // L):
            t = tgt_v[pl.ds(g * L, L)]
            row = base + g * L + iota
            idx_v[pl.ds(g * L, L)] = (
                (t >> 3) * (8 * (N // 128) * 128)
                + (row >> 7) * 1024
                + ((t & 7) << 7)
                + (row & 127)
            )

        gat_cp = pltpu.async_copy(prob_hbm.at[idx_v], vals_v, sem)
        rew_cp.wait()
        gat_cp.wait()

        acc = jnp.zeros((L,), jnp.float32)
        for g in range(rows_per_worker // L):
            acc = acc + vals_v[pl.ds(g * L, L)] * rew_v[pl.ds(g * L, L)]

        part_v[...] = acc
        pltpu.sync_copy(part_v, out_hbm.at[pl.ds(sid * L, L)])

    return sc_fn


def _tc_reduce(parts_ref, out_ref):
    out_ref[...] = -jnp.sum(parts_ref[...])[None, None]


@jax.jit
def kernel(prob, target, reward):
    N, C = prob.shape
    prob_flat = (
        prob.T.reshape(C // 8, 8, N // 128, 128)
        .transpose(0, 2, 1, 3)
        .reshape(N * C)
    )
    tgt = target.astype(jnp.int32)
    rew = reward.astype(jnp.float32)
    parts = _make_sc_call(N, C)(prob_flat, tgt, rew)
    out = pl.pallas_call(
        _tc_reduce,
        out_shape=jax.ShapeDtypeStruct((1, 1), jnp.float32),
    )(parts)
    return out[0, 0]


# speedup vs baseline: 1.0451x; 1.0451x over previous
import functools

import jax
import jax.numpy as jnp
from jax import lax
from jax.experimental import pallas as pl
from jax.experimental.pallas import tpu as pltpu
from jax.experimental.pallas import tpu_sc as plsc

L = 16


def _lane_shuffle(x, idx):
    dn = lax.GatherDimensionNumbers(
        offset_dims=(), collapsed_slice_dims=(0,), start_index_map=(0,)
    )
    return lax.gather(
        x, idx[:, None], dn, slice_sizes=(1,),
        mode=lax.GatherScatterMode.PROMISE_IN_BOUNDS,
    )


def _make_sc_call(N, C):
    assert C % L == 0 and N % L == 0
    chunks_per_row = C // L
    rows_per_worker = N // L

    mesh = plsc.VectorSubcoreMesh(
        core_axis_name="c", subcore_axis_name="s", num_cores=1, num_subcores=L
    )

    @functools.partial(
        pl.kernel,
        out_type=jax.ShapeDtypeStruct((L,), jnp.float32),
        mesh=mesh,
        scratch_types=[
            pltpu.VMEM((rows_per_worker,), jnp.int32),
            pltpu.VMEM((rows_per_worker,), jnp.float32),
            pltpu.VMEM((rows_per_worker,), jnp.int32),
            pltpu.VMEM((rows_per_worker,), jnp.float32),
            pltpu.VMEM((L,), jnp.float32),
            pltpu.VMEM_SHARED((L,), jnp.float32),
            pltpu.VMEM((L,), jnp.int32),
            pltpu.VMEM((L,), jnp.float32),
            pltpu.SemaphoreType.DMA,
            pltpu.SemaphoreType.DMA,
            pltpu.SemaphoreType.DMA,
        ],
        compiler_params=pltpu.CompilerParams(skip_device_barrier=True),
    )
    def sc_fn(prob_hbm, tgt_hbm, rew_hbm, out_hbm,
              tgt_v, rew_v, idx_v, vals_v, part_v,
              shared, idx16_v, out_v, sem, sem_t, sem_r):
        sid = lax.axis_index("s")
        base = sid * rows_per_worker

        tgt_cp = pltpu.async_copy(
            tgt_hbm.at[pl.ds(base, rows_per_worker)], tgt_v, sem_t)
        rew_cp = pltpu.async_copy(
            rew_hbm.at[pl.ds(base, rows_per_worker)], rew_v, sem_r)

        iota = lax.iota(jnp.int32, L)
        idx16_v[...] = iota

        @pl.when(sid == 0)
        def _():
            out_v[...] = jnp.zeros((L,), jnp.float32)
            pltpu.sync_copy(out_v, shared)

        plsc.subcore_barrier()
        tgt_cp.wait()
        gathers = []
        for g in range(rows_per_worker // L):
            t = tgt_v[pl.ds(g * L, L)]
            row = base + g * L + iota
            flat = (
                (t >> 3) * (8 * (N // 128) * 128)
                + (row >> 7) * 1024
                + ((t & 7) << 7)
                + (row & 127)
            )
            gathers.append(pltpu.async_copy(
                prob_hbm.at[flat], vals_v.at[pl.ds(g * L, L)], sem))
        rew_cp.wait()
        for cp in gathers:
            cp.wait()

        acc = jnp.zeros((L,), jnp.float32)
        for g in range(rows_per_worker // L):
            acc = acc + vals_v[pl.ds(g * L, L)] * rew_v[pl.ds(g * L, L)]

        part_v[...] = acc
        pltpu.sync_copy(part_v, shared.at[idx16_v], add=True)
        plsc.subcore_barrier()

        @pl.when(sid == 0)
        def _():
            pltpu.sync_copy(shared, part_v)
            tot = part_v[...]
            for sh in (1, 2, 4, 8):
                tot = tot + _lane_shuffle(tot, iota ^ sh)
            out_v[...] = -tot
            pltpu.sync_copy(out_v, out_hbm)

    return sc_fn


@jax.jit
def kernel(prob, target, reward):
    N, C = prob.shape
    prob_flat = (
        prob.T.reshape(C // 8, 8, N // 128, 128)
        .transpose(0, 2, 1, 3)
        .reshape(N * C)
    )
    tgt = target.astype(jnp.int32)
    rew = reward.astype(jnp.float32)
    out16 = _make_sc_call(N, C)(prob_flat, tgt, rew)
    return out16[0]


# speedup vs baseline: 1.0676x; 1.0216x over previous
import functools

import jax
import jax.numpy as jnp
from jax import lax
from jax._src.pallas import core as pallas_core
from jax.experimental import pallas as pl
from jax.experimental.pallas import tpu as pltpu
from jax.experimental.pallas import tpu_sc as plsc

L = 16


def _lane_shuffle(x, idx):
    dn = lax.GatherDimensionNumbers(
        offset_dims=(), collapsed_slice_dims=(0,), start_index_map=(0,)
    )
    return lax.gather(
        x, idx[:, None], dn, slice_sizes=(1,),
        mode=lax.GatherScatterMode.PROMISE_IN_BOUNDS,
    )


def _make_sc_call(N, C):
    assert C % 8 == 0 and N % 128 == 0 and N % L == 0
    rpw = N // L

    scalar_mesh = plsc.ScalarSubcoreMesh(axis_name="c", num_cores=1)
    vector_mesh = plsc.VectorSubcoreMesh(
        core_axis_name="c", subcore_axis_name="s", num_cores=1, num_subcores=L
    )
    tec_vmem = functools.partial(
        pallas_core.CoreMemorySpace, mesh=vector_mesh
    )(pltpu.VMEM)

    def scs_fn(prob_hbm, tgt_hbm, rew_hbm, out_hbm,
               tgt_sp, rew_sp, tgt_v, rew_v, idx_v, vals_v, part_v,
               shared, idx16_v, out_v, sem, sem_t, sem_r):
        pltpu.sync_copy(tgt_hbm, tgt_sp)
        pltpu.sync_copy(rew_hbm, rew_sp)

    def tec_fn(prob_hbm, tgt_hbm, rew_hbm, out_hbm,
               tgt_sp, rew_sp, tgt_v, rew_v, idx_v, vals_v, part_v,
               shared, idx16_v, out_v, sem, sem_t, sem_r):
        sid = lax.axis_index("s")
        base = sid * rpw

        tgt_cp = pltpu.async_copy(tgt_sp.at[pl.ds(base, rpw)], tgt_v, sem_t)
        rew_cp = pltpu.async_copy(rew_sp.at[pl.ds(base, rpw)], rew_v, sem_r)

        iota = lax.iota(jnp.int32, L)
        idx16_v[...] = iota

        @pl.when(sid == 0)
        def _():
            out_v[...] = jnp.zeros((L,), jnp.float32)
            pltpu.sync_copy(out_v, shared)

        plsc.subcore_barrier()
        tgt_cp.wait()
        for g in range(rpw // L):
            t = tgt_v[pl.ds(g * L, L)]
            row = base + g * L + iota
            idx_v[pl.ds(g * L, L)] = (
                (t >> 3) * (8 * (N // 128) * 128)
                + (row >> 7) * 1024
                + ((t & 7) << 7)
                + (row & 127)
            )

        gat_cp = pltpu.async_copy(prob_hbm.at[idx_v], vals_v, sem)
        rew_cp.wait()
        gat_cp.wait()

        acc = jnp.zeros((L,), jnp.float32)
        for g in range(rpw // L):
            acc = acc + vals_v[pl.ds(g * L, L)] * rew_v[pl.ds(g * L, L)]

        part_v[...] = acc
        pltpu.sync_copy(part_v, shared.at[idx16_v], add=True)
        plsc.subcore_barrier()

        @pl.when(sid == 0)
        def _():
            pltpu.sync_copy(shared, part_v)
            tot = part_v[...]
            for sh in (1, 2, 4, 8):
                tot = tot + _lane_shuffle(tot, iota ^ sh)
            out_v[...] = -tot
            pltpu.sync_copy(out_v, out_hbm)

    return pl.kernel(
        [scs_fn, tec_fn],
        out_type=jax.ShapeDtypeStruct((L,), jnp.float32),
        mesh=[scalar_mesh, vector_mesh],
        scratch_types=[
            pltpu.VMEM_SHARED((N,), jnp.int32),
            pltpu.VMEM_SHARED((N,), jnp.float32),
            tec_vmem((rpw,), jnp.int32),
            tec_vmem((rpw,), jnp.float32),
            tec_vmem((rpw,), jnp.int32),
            tec_vmem((rpw,), jnp.float32),
            tec_vmem((L,), jnp.float32),
            pltpu.VMEM_SHARED((L,), jnp.float32),
            tec_vmem((L,), jnp.int32),
            tec_vmem((L,), jnp.float32),
            pltpu.SemaphoreType.DMA @ vector_mesh,
            pltpu.SemaphoreType.DMA @ vector_mesh,
            pltpu.SemaphoreType.DMA @ vector_mesh,
        ],
        compiler_params=pltpu.CompilerParams(skip_device_barrier=True),
    )


@jax.jit
def kernel(prob, target, reward):
    N, C = prob.shape
    prob_flat = (
        prob.T.reshape(C // 8, 8, N // 128, 128)
        .transpose(0, 2, 1, 3)
        .reshape(N * C)
    )
    tgt = target.astype(jnp.int32)
    rew = reward.astype(jnp.float32)
    out16 = _make_sc_call(N, C)(prob_flat, tgt, rew)
    return out16[0]
